# Initial kernel scaffold; baseline (speedup 1.0000x reference)
import jax
import jax.numpy as jnp
from jax.experimental import pallas as pl


def kernel(constraint_features, edge_features, variable_features, edge_indices, task_id, cons_shift, cons_scale, cons_W1, cons_b1, cons_W2, cons_b2, edge_shift, edge_scale, edge_W, edge_b, var_shift, var_scale, var_W1, var_b1, var_W2, var_b2, vc_lw, vc_lb, vc_rw, vc_rb, vc_att, vc_ow1, vc_ob1, vc_ow2, vc_ob2, cv_lw, cv_lb, cv_rw, cv_rb, cv_att, cv_ow1, cv_ob1, cv_ow2, cv_ob2, o1_W1, o1_b1, o1_W2, o2_W1, o2_b1, o2_W2):
    raise NotImplementedError("write your pallas kernel here")


# speedup vs baseline: 1.4577x; 1.4577x over previous
import functools

import jax
import jax.numpy as jnp
from jax.experimental import pallas as pl

EMB = 64
HEADS = 8
OUTC = 8
NEG_SLOPE = 0.2


def _lrelu(x):
    return jnp.where(x >= 0, x, NEG_SLOPE * x)


def _gat_conv(left, src, dst, s_e, right, lw, lb, rw, rb, att, ow1, ob1, ow2, ob2):
    x_l = left @ lw + lb
    x_r = right @ rw + rb
    attm = att.reshape(HEADS * 3 * OUTC)
    a_i = attm[:EMB]
    a_j = attm[EMB:2 * EMB]
    seg_i = jnp.arange(EMB) // (3 * OUTC)
    seg_j = (jnp.arange(EMB) + EMB) // (3 * OUTC)
    hot_i = (seg_i[:, None] == jnp.arange(HEADS)[None, :]).astype(jnp.float32)
    hot_j = (seg_j[:, None] == jnp.arange(HEADS)[None, :]).astype(jnp.float32)
    s_dst = jnp.dot(_lrelu(x_r), a_i[:, None] * hot_i,
                    precision=jax.lax.Precision.HIGHEST)
    s_src = jnp.dot(_lrelu(x_l), a_j[:, None] * hot_j,
                    precision=jax.lax.Precision.HIGHEST)

    raw = s_dst[dst] + s_src[src] + s_e
    gmax = (jnp.max(s_dst, axis=0) + jnp.max(s_src, axis=0)
            + jnp.max(s_e, axis=0))
    alpha = jnp.exp(raw - gmax)
    n_r = right.shape[0]
    asum = jax.ops.segment_sum(alpha, dst, num_segments=n_r)
    w = alpha / asum[dst]
    msg = x_l[src].reshape(-1, HEADS, OUTC) * w[:, :, None]
    agg = jax.ops.segment_sum(msg.reshape(-1, HEADS * OUTC), dst, num_segments=n_r)
    h = jnp.concatenate([agg, right], axis=-1)
    h = jax.nn.relu(h @ ow1 + ob1)
    return h @ ow2 + ob2


def _out_mlp_kernel(v_ref, w1_ref, b1_ref, w2_ref, o_ref):
    h = jnp.maximum(v_ref[...] @ w1_ref[...] + b1_ref[...], 0.0)
    o_ref[...] = h @ w2_ref[...]


def _out_mlp(v, W1, b1, W2):
    n = v.shape[0]
    blk = 2000
    out = pl.pallas_call(
        _out_mlp_kernel,
        grid=(n // blk,),
        in_specs=[
            pl.BlockSpec((blk, EMB), lambda i: (i, 0)),
            pl.BlockSpec((EMB, EMB), lambda i: (0, 0)),
            pl.BlockSpec((1, EMB), lambda i: (0, 0)),
            pl.BlockSpec((EMB, 128), lambda i: (0, 0)),
        ],
        out_specs=pl.BlockSpec((blk, 128), lambda i: (i, 0)),
        out_shape=jax.ShapeDtypeStruct((n, 128), jnp.float32),
    )(v, W1, b1.reshape(1, EMB), W2)
    return out[:, 0]


def kernel(constraint_features, edge_features, variable_features, edge_indices, task_id, cons_shift, cons_scale, cons_W1, cons_b1, cons_W2, cons_b2, edge_shift, edge_scale, edge_W, edge_b, var_shift, var_scale, var_W1, var_b1, var_W2, var_b2, vc_lw, vc_lb, vc_rw, vc_rb, vc_att, vc_ow1, vc_ob1, vc_ow2, vc_ob2, cv_lw, cv_lb, cv_rw, cv_rb, cv_att, cv_ow1, cv_ob1, cv_ow2, cv_ob2, o1_W1, o1_b1, o1_W2, o2_W1, o2_b1, o2_W2):
    c = (constraint_features + cons_shift) * cons_scale
    c = jax.nn.relu(c @ cons_W1 + cons_b1)
    c = jax.nn.relu(c @ cons_W2 + cons_b2)
    v = (variable_features + var_shift) * var_scale
    v = jax.nn.relu(v @ var_W1 + var_b1)
    v = jax.nn.relu(v @ var_W2 + var_b2)

    ef = ((edge_features + edge_shift) * edge_scale)[:, 0]

    def edge_scores(att):
        attm = att.reshape(HEADS * 3 * OUTC)[2 * EMB:]
        seg = (jnp.arange(EMB) + 2 * EMB) // (3 * OUTC)
        hot = (seg[:, None] == jnp.arange(HEADS)[None, :]).astype(jnp.float32)
        emb = ef[:, None] * edge_W[0][None, :] + edge_b[None, :]
        return jnp.dot(_lrelu(emb), attm[:, None] * hot,
                       precision=jax.lax.Precision.HIGHEST)

    s_e_vc = edge_scores(vc_att)
    s_e_cv = edge_scores(cv_att)

    ci = edge_indices[0]
    vi = edge_indices[1]
    c = _gat_conv(v, vi, ci, s_e_vc, c, vc_lw, vc_lb, vc_rw, vc_rb, vc_att,
                  vc_ow1, vc_ob1, vc_ow2, vc_ob2)
    v = _gat_conv(c, ci, vi, s_e_cv, v, cv_lw, cv_lb, cv_rw, cv_rb, cv_att,
                  cv_ow1, cv_ob1, cv_ow2, cv_ob2)

    W1 = jnp.where(task_id == 0, o1_W1, o2_W1)
    b1 = jnp.where(task_id == 0, o1_b1, o2_b1)
    W2 = jnp.where(task_id == 0, o1_W2, o2_W2)
    W2p = jnp.pad(W2, ((0, 0), (0, 127)))
    return _out_mlp(v, W1, b1, W2p)


# speedup vs baseline: 2.7475x; 1.8849x over previous
import functools

import jax
import jax.numpy as jnp
from jax import lax
from jax.experimental import pallas as pl
from jax.experimental.pallas import tpu as pltpu
from jax.experimental.pallas import tpu_sc as plsc

EMB = 64
HEADS = 8
OUTC = 8
NEG_SLOPE = 0.2
HI = jax.lax.Precision.HIGHEST

N_NODES = 50000
NP = 51200
N_EDGES = 800000
EP = 819200
NW = 32
CH = 512
CPW = EP // NW // CH

NR_SC = 25600
AGG_ROWS = 26624
CH2 = 128

NBLK = 2000
EBLK = 6400


def _sc_mesh():
    return plsc.VectorSubcoreMesh(core_axis_name="c", subcore_axis_name="s",
                                  num_cores=2, num_subcores=16)


def _p1_body(src_hbm, dst_hbm, sd_hbm, se_hbm,
             alpha_hbm, asum_hbm,
             idx_s, idx_d, g1_b, g2_b, se_b, alpha_b, asum_sh, zeros_b,
             sem):
    cid = lax.axis_index("c")
    sid = lax.axis_index("s")
    wid = cid * 16 + sid

    def zb(i, _):
        zeros_b[i, :] = jnp.zeros((16,), jnp.float32)
        return _

    lax.fori_loop(0, 128, zb, 0, unroll=8)

    def zclear(i, _):
        pltpu.sync_copy(zeros_b, asum_sh.at[pl.ds(sid * 3200 + i * 128, 128)])
        return _

    lax.fori_loop(0, 25, zclear, 0)
    plsc.subcore_barrier()

    def chunk(ci, _):
        row0 = wid * (CPW * 4) + ci * 4
        e0 = row0 * 128
        pltpu.sync_copy(src_hbm.at[pl.ds(row0, 4)], idx_s)
        pltpu.sync_copy(dst_hbm.at[pl.ds(row0, 4)], idx_d)
        pltpu.sync_copy(se_hbm.at[pl.ds(e0, CH)], se_b)
        for j in range(4):
            pltpu.async_copy(sd_hbm.at[idx_s.at[j]],
                             g1_b.at[pl.ds(j * 128, 128)], sem).wait()
            pltpu.async_copy(sd_hbm.at[idx_d.at[j]],
                             g2_b.at[pl.ds(j * 128, 128)], sem).wait()

        def inner(i, _):
            a = g1_b[i, :]
            b = jnp.flip(g2_b[i, :])
            c = se_b[i, :]
            alpha_b[i, :] = jnp.exp(jnp.minimum(a + b + c, 0.0))
            return _

        lax.fori_loop(0, CH, inner, 0, unroll=8)

        pltpu.sync_copy(alpha_b, alpha_hbm.at[pl.ds(e0, CH)])
        for j in range(4):
            pltpu.sync_copy(alpha_b.at[pl.ds(j * 128, 128)],
                            asum_sh.at[idx_d.at[j]], add=True)
        return _

    lax.fori_loop(0, CPW, chunk, 0)
    plsc.subcore_barrier()
    pltpu.sync_copy(asum_sh.at[pl.ds(sid * 3200, 3200)],
                    asum_hbm.at[cid].at[pl.ds(sid * 3200, 3200)])


def _sc_p1(src2d, dst2d, sd_packed, se16):
    k = pl.kernel(
        _p1_body,
        out_type=(jax.ShapeDtypeStruct((EP, 16), jnp.float32),
                  jax.ShapeDtypeStruct((2, NP, 16), jnp.float32)),
        mesh=_sc_mesh(),
        scratch_types=[
            pltpu.VMEM((4, 128), jnp.int32),
            pltpu.VMEM((4, 128), jnp.int32),
            pltpu.VMEM((CH, 16), jnp.float32),
            pltpu.VMEM((CH, 16), jnp.float32),
            pltpu.VMEM((CH, 16), jnp.float32),
            pltpu.VMEM((CH, 16), jnp.float32),
            pltpu.VMEM_SHARED((NP, 16), jnp.float32),
            pltpu.VMEM((128, 16), jnp.float32),
            pltpu.SemaphoreType.DMA,
        ],
        compiler_params=pltpu.CompilerParams(use_tc_tiling_on_sc=False),
    )
    return k(src2d, dst2d, sd_packed, se16)


def _p2_body(src_hbm, dst_hbm, alpha_hbm, rsum_hbm, xlt_hbm,
             agg_hbm,
             idx_s, idx_d, sidx, al_b, rg_b, xl_b, msg_b, agg_sh,
             sem):
    cid = lax.axis_index("c")
    sid = lax.axis_index("s")
    base = cid * NR_SC

    def zb(i, _):
        for k in range(4):
            msg_b[i, pl.ds(k * 16, 16)] = jnp.zeros((16,), jnp.float32)
        return _

    lax.fori_loop(0, 128, zb, 0, unroll=8)

    def zclear(i, _):
        pltpu.sync_copy(msg_b, agg_sh.at[pl.ds(sid * 1664 + i * 128, 128)])
        return _

    lax.fori_loop(0, 13, zclear, 0)
    plsc.subcore_barrier()

    lane = lax.iota(jnp.int32, 16)
    low = lane < 8

    def chunk(ci, _):
        row0 = sid * (EP // 16 // 128) + ci
        e0 = row0 * 128
        pltpu.sync_copy(src_hbm.at[pl.ds(row0, 1)], idx_s)
        pltpu.sync_copy(dst_hbm.at[pl.ds(row0, 1)], idx_d)
        pltpu.sync_copy(alpha_hbm.at[pl.ds(e0, CH2)], al_b)
        pltpu.async_copy(rsum_hbm.at[idx_d.at[0]], rg_b, sem).wait()
        pltpu.async_copy(xlt_hbm.at[idx_s.at[0]], xl_b, sem).wait()
        for k in range(8):
            d = idx_d[0, pl.ds(k * 16, 16)]
            l = d - base
            inb = (l >= 0) & (l < NR_SC)
            sidx[0, pl.ds(k * 16, 16)] = jnp.where(inb, l, NR_SC)

        def inner(i, _):
            w = al_b[i, :] * rg_b[i, :]
            wp = jnp.where(low, w, jnp.flip(w))
            for k in range(4):
                msg_b[i, pl.ds(k * 16, 16)] = xl_b[i, pl.ds(k * 16, 16)] * wp
            return _

        lax.fori_loop(0, CH2, inner, 0, unroll=4)

        pltpu.sync_copy(msg_b, agg_sh.at[sidx.at[0]], add=True)
        return _

    lax.fori_loop(0, EP // 16 // CH2, chunk, 0)
    plsc.subcore_barrier()
    pltpu.sync_copy(agg_sh.at[pl.ds(sid * 1600, 1600)],
                    agg_hbm.at[cid].at[pl.ds(sid * 1600, 1600)])


def _sc_p2(src2d, dst2d, alpha16, rsum16, xlt):
    k = pl.kernel(
        _p2_body,
        out_type=jax.ShapeDtypeStruct((2, NR_SC, EMB), jnp.float32),
        mesh=_sc_mesh(),
        scratch_types=[
            pltpu.VMEM((1, 128), jnp.int32),
            pltpu.VMEM((1, 128), jnp.int32),
            pltpu.VMEM((1, 128), jnp.int32),
            pltpu.VMEM((CH2, 16), jnp.float32),
            pltpu.VMEM((CH2, 16), jnp.float32),
            pltpu.VMEM((CH2, EMB), jnp.float32),
            pltpu.VMEM((CH2, EMB), jnp.float32),
            pltpu.VMEM_SHARED((AGG_ROWS, EMB), jnp.float32),
            pltpu.SemaphoreType.DMA,
        ],
        compiler_params=pltpu.CompilerParams(use_tc_tiling_on_sc=False),
    )
    return k(src2d, dst2d, alpha16, rsum16, xlt)


def _shuffle_perm():
    perm = [0] * 64
    for c in range(64):
        s, h = c & 7, c >> 3
        k = s >> 1
        perm[c] = 16 * k + (h if s % 2 == 0 else 15 - h)
    return perm


_PERM = _shuffle_perm()
_INV_PERM = [0] * 64
for _c, _p in enumerate(_PERM):
    _INV_PERM[_p] = _c


def _lrelu(x):
    return jnp.where(x >= 0, x, NEG_SLOPE * x)


def _mxacc(mx_ref, s_val):
    bm = jnp.max(s_val, axis=0, keepdims=True)

    @pl.when(pl.program_id(0) == 0)
    def _():
        mx_ref[...] = bm

    @pl.when(pl.program_id(0) != 0)
    def _():
        mx_ref[...] = jnp.maximum(mx_ref[...], bm)


def _k1_body(f, w1, b1, w2, b2, rw, rb, mi, emb_o, s_o, mx_o):
    h = jnp.maximum(jnp.dot(f[...], w1[...]) + b1[...], 0.0)
    emb = jnp.maximum(jnp.dot(h, w2[...]) + b2[...], 0.0)
    emb_o[...] = emb
    x = jnp.dot(emb, rw[...]) + rb[...]
    s = jnp.dot(_lrelu(x), mi[...], precision=HI)
    s_o[...] = s
    _mxacc(mx_o, s)


def _tc_k1(f, w1, b1, w2, b2, rw, rb, mi):
    n, fdim = f.shape
    z = lambda i: (0, 0)
    return pl.pallas_call(
        _k1_body,
        grid=(n // NBLK,),
        in_specs=[
            pl.BlockSpec((NBLK, fdim), lambda i: (i, 0)),
            pl.BlockSpec((fdim, EMB), z), pl.BlockSpec((1, EMB), z),
            pl.BlockSpec((EMB, EMB), z), pl.BlockSpec((1, EMB), z),
            pl.BlockSpec((EMB, EMB), z), pl.BlockSpec((1, EMB), z),
            pl.BlockSpec((EMB, HEADS), z),
        ],
        out_specs=[
            pl.BlockSpec((NBLK, EMB), lambda i: (i, 0)),
            pl.BlockSpec((NBLK, HEADS), lambda i: (i, 0)),
            pl.BlockSpec((1, HEADS), z),
        ],
        out_shape=[
            jax.ShapeDtypeStruct((n, EMB), jnp.float32),
            jax.ShapeDtypeStruct((n, HEADS), jnp.float32),
            jax.ShapeDtypeStruct((1, HEADS), jnp.float32),
        ],
    )(f, w1, b1.reshape(1, -1), w2, b2.reshape(1, -1), rw, rb.reshape(1, -1),
      mi)


def _k2_body(f, w1, b1, w2, b2, lwt, lbt, mjt, rw, rb, mi,
             emb_o, xlt_o, ss_o, mxs_o, sd_o, mxd_o):
    h = jnp.maximum(jnp.dot(f[...], w1[...]) + b1[...], 0.0)
    emb = jnp.maximum(jnp.dot(h, w2[...]) + b2[...], 0.0)
    emb_o[...] = emb
    xlt = jnp.dot(emb, lwt[...]) + lbt[...]
    xlt_o[...] = xlt
    ss = jnp.dot(_lrelu(xlt), mjt[...], precision=HI)
    ss_o[...] = ss
    _mxacc(mxs_o, ss)
    xr = jnp.dot(emb, rw[...]) + rb[...]
    sd = jnp.dot(_lrelu(xr), mi[...], precision=HI)
    sd_o[...] = sd
    _mxacc(mxd_o, sd)


def _tc_k2(f, w1, b1, w2, b2, lwt, lbt, mjt, rw, rb, mi):
    n, fdim = f.shape
    z = lambda i: (0, 0)
    return pl.pallas_call(
        _k2_body,
        grid=(n // NBLK,),
        in_specs=[
            pl.BlockSpec((NBLK, fdim), lambda i: (i, 0)),
            pl.BlockSpec((fdim, EMB), z), pl.BlockSpec((1, EMB), z),
            pl.BlockSpec((EMB, EMB), z), pl.BlockSpec((1, EMB), z),
            pl.BlockSpec((EMB, EMB), z), pl.BlockSpec((1, EMB), z),
            pl.BlockSpec((EMB, HEADS), z),
            pl.BlockSpec((EMB, EMB), z), pl.BlockSpec((1, EMB), z),
            pl.BlockSpec((EMB, HEADS), z),
        ],
        out_specs=[
            pl.BlockSpec((NBLK, EMB), lambda i: (i, 0)),
            pl.BlockSpec((NBLK, EMB), lambda i: (i, 0)),
            pl.BlockSpec((NBLK, HEADS), lambda i: (i, 0)),
            pl.BlockSpec((1, HEADS), z),
            pl.BlockSpec((NBLK, HEADS), lambda i: (i, 0)),
            pl.BlockSpec((1, HEADS), z),
        ],
        out_shape=[
            jax.ShapeDtypeStruct((n, EMB), jnp.float32),
            jax.ShapeDtypeStruct((n, EMB), jnp.float32),
            jax.ShapeDtypeStruct((n, HEADS), jnp.float32),
            jax.ShapeDtypeStruct((1, HEADS), jnp.float32),
            jax.ShapeDtypeStruct((n, HEADS), jnp.float32),
            jax.ShapeDtypeStruct((1, HEADS), jnp.float32),
        ],
    )(f, w1, b1.reshape(1, -1), w2, b2.reshape(1, -1), lwt, lbt.reshape(1, -1),
      mjt, rw, rb.reshape(1, -1), mi)


def _k3_body(ef, ew, eb, mevc, mecv, svc_o, scv_o, mxvc_o, mxcv_o):
    emb = ef[...] * ew[...] + eb[...]
    lr = _lrelu(emb)
    zer = jnp.zeros((lr.shape[0], HEADS), jnp.float32)
    svc = jnp.dot(lr, mevc[...], precision=HI)
    scv = jnp.dot(lr, mecv[...], precision=HI)
    svc_o[...] = jnp.concatenate([svc, zer], axis=1)
    scv_o[...] = jnp.concatenate([scv, zer], axis=1)
    _mxacc(mxvc_o, svc)
    _mxacc(mxcv_o, scv)


def _tc_k3(efp, ew, eb, mevc, mecv):
    n = efp.shape[0]
    z = lambda i: (0, 0)
    return pl.pallas_call(
        _k3_body,
        grid=(n // EBLK,),
        in_specs=[
            pl.BlockSpec((EBLK, 1), lambda i: (i, 0)),
            pl.BlockSpec((1, EMB), z), pl.BlockSpec((1, EMB), z),
            pl.BlockSpec((EMB, HEADS), z), pl.BlockSpec((EMB, HEADS), z),
        ],
        out_specs=[
            pl.BlockSpec((EBLK, 16), lambda i: (i, 0)),
            pl.BlockSpec((EBLK, 16), lambda i: (i, 0)),
            pl.BlockSpec((1, HEADS), z), pl.BlockSpec((1, HEADS), z),
        ],
        out_shape=[
            jax.ShapeDtypeStruct((n, 16), jnp.float32),
            jax.ShapeDtypeStruct((n, 16), jnp.float32),
            jax.ShapeDtypeStruct((1, HEADS), jnp.float32),
            jax.ShapeDtypeStruct((1, HEADS), jnp.float32),
        ],
    )(efp, ew, eb.reshape(1, -1), mevc, mecv)


def _k4_body(aggt, right, w1a, w1b, ob1, ow2, ob2, lwt, lbt, mjt,
             xlt_o, ss_o, mxs_o):
    h = jnp.maximum(jnp.dot(aggt[...], w1a[...])
                    + jnp.dot(right[...], w1b[...])
                    + ob1[...], 0.0)
    c1 = jnp.dot(h, ow2[...]) + ob2[...]
    xlt = jnp.dot(c1, lwt[...]) + lbt[...]
    xlt_o[...] = xlt
    ss = jnp.dot(_lrelu(xlt), mjt[...], precision=HI)
    ss_o[...] = ss
    _mxacc(mxs_o, ss)


def _tc_k4(aggt, right, w1a, w1b, ob1, ow2, ob2, lwt, lbt, mjt):
    n = aggt.shape[0]
    z = lambda i: (0, 0)
    return pl.pallas_call(
        _k4_body,
        grid=(n // NBLK,),
        in_specs=[
            pl.BlockSpec((NBLK, EMB), lambda i: (i, 0)),
            pl.BlockSpec((NBLK, EMB), lambda i: (i, 0)),
            pl.BlockSpec((EMB, EMB), z), pl.BlockSpec((EMB, EMB), z),
            pl.BlockSpec((1, EMB), z),
            pl.BlockSpec((EMB, EMB), z), pl.BlockSpec((1, EMB), z),
            pl.BlockSpec((EMB, EMB), z), pl.BlockSpec((1, EMB), z),
            pl.BlockSpec((EMB, HEADS), z),
        ],
        out_specs=[
            pl.BlockSpec((NBLK, EMB), lambda i: (i, 0)),
            pl.BlockSpec((NBLK, HEADS), lambda i: (i, 0)),
            pl.BlockSpec((1, HEADS), z),
        ],
        out_shape=[
            jax.ShapeDtypeStruct((n, EMB), jnp.float32),
            jax.ShapeDtypeStruct((n, HEADS), jnp.float32),
            jax.ShapeDtypeStruct((1, HEADS), jnp.float32),
        ],
    )(aggt, right, w1a, w1b, ob1.reshape(1, -1), ow2, ob2.reshape(1, -1),
      lwt, lbt.reshape(1, -1), mjt)


def _k5_body(aggt, right, w1a, w1b, ob1, ow2, ob2, hw1, hb1, hw2, o_ref):
    h = jnp.maximum(jnp.dot(aggt[...], w1a[...])
                    + jnp.dot(right[...], w1b[...])
                    + ob1[...], 0.0)
    v1 = jnp.dot(h, ow2[...]) + ob2[...]
    h2 = jnp.maximum(jnp.dot(v1, hw1[...]) + hb1[...], 0.0)
    o_ref[...] = jnp.dot(h2, hw2[...])


def _tc_k5(aggt, right, w1a, w1b, ob1, ow2, ob2, hw1, hb1, hw2pad):
    n = aggt.shape[0]
    z = lambda i: (0, 0)
    return pl.pallas_call(
        _k5_body,
        grid=(n // NBLK,),
        in_specs=[
            pl.BlockSpec((NBLK, EMB), lambda i: (i, 0)),
            pl.BlockSpec((NBLK, EMB), lambda i: (i, 0)),
            pl.BlockSpec((EMB, EMB), z), pl.BlockSpec((EMB, EMB), z),
            pl.BlockSpec((1, EMB), z),
            pl.BlockSpec((EMB, EMB), z), pl.BlockSpec((1, EMB), z),
            pl.BlockSpec((EMB, EMB), z), pl.BlockSpec((1, EMB), z),
            pl.BlockSpec((EMB, 8), z),
        ],
        out_specs=pl.BlockSpec((NBLK, 8), lambda i: (i, 0)),
        out_shape=jax.ShapeDtypeStruct((n, 8), jnp.float32),
    )(aggt, right, w1a, w1b, ob1.reshape(1, -1), ow2, ob2.reshape(1, -1),
      hw1, hb1.reshape(1, -1), hw2pad)


def _att_mats(att):
    attm = att.reshape(HEADS * 3 * OUTC)
    heads = jnp.arange(HEADS)[None, :]

    def mat(lo):
        w = attm[lo:lo + EMB]
        seg = (jnp.arange(EMB) + lo) // (3 * OUTC)
        hot = (seg[:, None] == heads).astype(jnp.float32)
        return w[:, None] * hot

    return mat(0), mat(EMB), mat(2 * EMB)


def _pad_rows(x, rows):
    return jnp.zeros((rows,) + x.shape[1:], x.dtype).at[:x.shape[0]].set(x)


def _conv_sc(src2d, dst2d, sd_packed, se16, rsum_of, xlt):
    alpha16, asum_parts = _sc_p1(src2d, dst2d, sd_packed, se16)
    asum = (asum_parts[0] + asum_parts[1])[:N_NODES, :HEADS]
    rsum16 = jnp.zeros((NP, 16), jnp.float32).at[:N_NODES, :HEADS].set(
        1.0 / asum)
    agg_parts = _sc_p2(src2d, dst2d, alpha16, rsum16, xlt)
    return agg_parts.reshape(2 * NR_SC, EMB)[:N_NODES]


def kernel(constraint_features, edge_features, variable_features, edge_indices, task_id, cons_shift, cons_scale, cons_W1, cons_b1, cons_W2, cons_b2, edge_shift, edge_scale, edge_W, edge_b, var_shift, var_scale, var_W1, var_b1, var_W2, var_b2, vc_lw, vc_lb, vc_rw, vc_rb, vc_att, vc_ow1, vc_ob1, vc_ow2, vc_ob2, cv_lw, cv_lb, cv_rw, cv_rb, cv_att, cv_ow1, cv_ob1, cv_ow2, cv_ob2, o1_W1, o1_b1, o1_W2, o2_W1, o2_b1, o2_W2):
    inv = jnp.array(_INV_PERM)

    cW1 = cons_scale[:, None] * cons_W1
    cb1 = cons_b1 + (cons_shift * cons_scale) @ cons_W1
    vW1 = var_scale[:, None] * var_W1
    vb1 = var_b1 + (var_shift * var_scale) @ var_W1
    eW = edge_scale[0] * edge_W
    eb = edge_b + (edge_shift * edge_scale) @ edge_W

    mi_vc, mj_vc, me_vc = _att_mats(vc_att)
    mi_cv, mj_cv, me_cv = _att_mats(cv_att)

    c0, sdst_vc, mx_dvc = _tc_k1(constraint_features, cW1, cb1,
                                 cons_W2, cons_b2, vc_rw, vc_rb, mi_vc)
    v0, xlt_vc, ssrc_vc, mx_svc, sdst_cv, mx_dcv = _tc_k2(
        variable_features, vW1, vb1, var_W2, var_b2,
        vc_lw[:, inv], vc_lb[inv], jnp.asarray(mj_vc)[inv],
        cv_rw, cv_rb, mi_cv)
    efp = _pad_rows(edge_features, EP)
    se_vc, se_cv, mx_evc, mx_ecv = _tc_k3(efp, eW, eb, me_vc, me_cv)

    ci = edge_indices[0]
    vi = edge_indices[1]
    pad_e = EP - N_EDGES
    ci2d = jnp.concatenate(
        [ci, jnp.full((pad_e,), N_NODES, jnp.int32)]).reshape(EP // 128, 128)
    vi2d = jnp.concatenate(
        [vi, jnp.full((pad_e,), N_NODES, jnp.int32)]).reshape(EP // 128, 128)

    gmax_vc = (mx_svc + mx_dvc + mx_evc)[0]
    sd_vc = _pad_rows(jnp.concatenate(
        [ssrc_vc, (sdst_vc - gmax_vc)[:, ::-1]], axis=1), NP)
    agg_vc = _conv_sc(vi2d, ci2d, sd_vc, se_vc, None, _pad_rows(xlt_vc, NP))

    xlt_cv, ssrc_cv, mx_scv = _tc_k4(
        agg_vc, c0, vc_ow1[:EMB][inv], vc_ow1[EMB:], vc_ob1, vc_ow2, vc_ob2,
        cv_lw[:, inv], cv_lb[inv], jnp.asarray(mj_cv)[inv])

    gmax_cv = (mx_scv + mx_dcv + mx_ecv)[0]
    sd_cv = _pad_rows(jnp.concatenate(
        [ssrc_cv, (sdst_cv - gmax_cv)[:, ::-1]], axis=1), NP)
    agg_cv = _conv_sc(ci2d, vi2d, sd_cv, se_cv, None, _pad_rows(xlt_cv, NP))

    W1 = jnp.where(task_id == 0, o1_W1, o2_W1)
    b1 = jnp.where(task_id == 0, o1_b1, o2_b1)
    W2 = jnp.where(task_id == 0, o1_W2, o2_W2)
    out8 = _tc_k5(agg_cv, v0, cv_ow1[:EMB][inv], cv_ow1[EMB:], cv_ob1,
                  cv_ow2, cv_ob2, W1, b1, jnp.pad(W2, ((0, 0), (0, 7))))
    return out8[:, 0]
